# Initial kernel scaffold; baseline (speedup 1.0000x reference)
import jax
import jax.numpy as jnp
from jax.experimental import pallas as pl


def kernel(adj_edge_index, adj_values, embeds):
    raise NotImplementedError("write your pallas kernel here")


# speedup vs baseline: 4.0366x; 4.0366x over previous
import functools

import jax
import jax.numpy as jnp
from jax import lax
from jax.experimental import pallas as pl
from jax.experimental.pallas import tpu as pltpu
from jax.experimental.pallas import tpu_sc as plsc

N_NODES = 10000
N_EDGES = 320000
D_FEAT = 128

NC = 2
NS = 16
NW = NC * NS
DH = D_FEAT // NC

CHUNK = 80
EPT = N_EDGES // NS
NCHUNK = EPT // CHUNK
ROW_STEP = 624
ROW_SPAN = 640
NLANE = DH // 16


def _sc_body(row_hbm, col_hbm, val_hbm, emb_hbm, out_hbm,
             row_v, col_v, val_v, gbuf, acc, sem):
    c = lax.axis_index("c")
    s = lax.axis_index("s")

    pltpu.sync_copy(row_hbm.at[s], row_v)
    pltpu.sync_copy(col_hbm.at[s], col_v)
    pltpu.sync_copy(val_hbm.at[pl.ds(s * EPT, EPT)], val_v.at[pl.ds(0, EPT)])

    cofs = jnp.full((16,), c * N_NODES, jnp.int32)

    def _bias(i, _):
        for t in range(CHUNK // 16):
            col_v[i, pl.ds(t * 16, 16)] = col_v[i, pl.ds(t * 16, 16)] + cofs
        return 0
    lax.fori_loop(0, NCHUNK, _bias, 0)

    def _zrow(i, _):
        for j in range(NLANE):
            gbuf[i, pl.ds(j * 16, 16)] = jnp.zeros((16,), jnp.float32)
        return 0
    lax.fori_loop(0, CHUNK, _zrow, 0)
    zbase = s * ROW_STEP
    for k in range(ROW_SPAN // CHUNK):
        pltpu.sync_copy(gbuf, acc.at[pl.ds(zbase + k * CHUNK, CHUNK)])
    plsc.subcore_barrier()

    def _chunk(i, _):
        pltpu.async_copy(emb_hbm.at[col_v.at[i]], gbuf, sem).wait()

        def _edge(e, _):
            vv = val_v[pl.ds(i * CHUNK + e, 16)]
            vs = jnp.full((16,), vv[0], jnp.float32)
            for j in range(NLANE):
                gbuf[e, pl.ds(j * 16, 16)] = gbuf[e, pl.ds(j * 16, 16)] * vs
            return 0
        lax.fori_loop(0, CHUNK, _edge, 0)

        pltpu.sync_copy(gbuf, acc.at[row_v.at[i]], add=True)
        return 0
    lax.fori_loop(0, NCHUNK, _chunk, 0)
    plsc.subcore_barrier()

    pltpu.sync_copy(acc.at[pl.ds(s * ROW_STEP, ROW_SPAN)],
                    out_hbm.at[c, pl.ds(s * ROW_STEP, ROW_SPAN)])


_sc_spmm = functools.partial(
    pl.kernel,
    out_type=jax.ShapeDtypeStruct((NC, N_NODES, DH), jnp.float32),
    mesh=plsc.VectorSubcoreMesh(core_axis_name="c", subcore_axis_name="s",
                                num_cores=NC, num_subcores=NS),
    scratch_types=[
        pltpu.VMEM((NCHUNK, CHUNK), jnp.int32),
        pltpu.VMEM((NCHUNK, CHUNK), jnp.int32),
        pltpu.VMEM((EPT + 16,), jnp.float32),
        pltpu.VMEM((CHUNK, DH), jnp.float32),
        pltpu.VMEM_SHARED((N_NODES, DH), jnp.float32),
        pltpu.SemaphoreType.DMA,
    ],
    compiler_params=pltpu.CompilerParams(use_tc_tiling_on_sc=False),
)(_sc_body)


def kernel(adj_edge_index, adj_values, embeds):
    row = adj_edge_index[0].reshape(NS, NCHUNK, CHUNK)
    col = adj_edge_index[1].reshape(NS, NCHUNK, CHUNK)
    emb = embeds.reshape(N_NODES, NC, DH).transpose(1, 0, 2).reshape(NC * N_NODES, DH)
    out = _sc_spmm(row, col, adj_values, emb)
    return out.transpose(1, 0, 2).reshape(N_NODES, D_FEAT)


# speedup vs baseline: 4.5874x; 1.1364x over previous
import functools

import jax
import jax.numpy as jnp
from jax import lax
from jax.experimental import pallas as pl
from jax.experimental.pallas import tpu as pltpu
from jax.experimental.pallas import tpu_sc as plsc

N_NODES = 10000
N_EDGES = 320000
D_FEAT = 128

NC = 2
NS = 16
NW = NC * NS
DH = D_FEAT // NC

CHUNK = 80
EPT = N_EDGES // NS
NCHUNK = EPT // CHUNK
ROW_STEP = 624
ROW_SPAN = 640
NLANE = DH // 16


def _sc_body(row_hbm, col_hbm, val_hbm, emb_hbm, out_hbm,
             row_v, col_v, val_v, gbuf, acc, sem):
    c = lax.axis_index("c")
    s = lax.axis_index("s")

    pltpu.sync_copy(row_hbm.at[s], row_v)
    pltpu.sync_copy(col_hbm.at[s], col_v)
    pltpu.sync_copy(val_hbm.at[pl.ds(s * EPT, EPT)], val_v.at[pl.ds(0, EPT)])

    cofs = jnp.full((16,), c * N_NODES, jnp.int32)

    def _bias(i, _):
        for t in range(CHUNK // 16):
            col_v[i, pl.ds(t * 16, 16)] = col_v[i, pl.ds(t * 16, 16)] + cofs
        return 0
    lax.fori_loop(0, NCHUNK, _bias, 0)

    def _zrow(i, _):
        for j in range(NLANE):
            gbuf[i, pl.ds(j * 16, 16)] = jnp.zeros((16,), jnp.float32)
        return 0
    lax.fori_loop(0, CHUNK, _zrow, 0)
    zbase = s * ROW_STEP
    for k in range(ROW_SPAN // CHUNK):
        pltpu.sync_copy(gbuf, acc.at[pl.ds(zbase + k * CHUNK, CHUNK)])
    plsc.subcore_barrier()

    def _chunk(i, _):
        pltpu.async_copy(emb_hbm.at[col_v.at[i]], gbuf, sem).wait()

        def _group(g, _):
            e0 = g * 16
            vv = val_v[pl.ds(i * CHUNK + e0, 16)]
            for el in range(16):
                vs = jnp.full((16,), vv[el], jnp.float32)
                for j in range(NLANE):
                    gbuf[e0 + el, pl.ds(j * 16, 16)] = (
                        gbuf[e0 + el, pl.ds(j * 16, 16)] * vs)
            return 0
        lax.fori_loop(0, CHUNK // 16, _group, 0)

        pltpu.sync_copy(gbuf, acc.at[row_v.at[i]], add=True)
        return 0
    lax.fori_loop(0, NCHUNK, _chunk, 0)
    plsc.subcore_barrier()

    pltpu.sync_copy(acc.at[pl.ds(s * ROW_STEP, ROW_SPAN)],
                    out_hbm.at[c, pl.ds(s * ROW_STEP, ROW_SPAN)])


_sc_spmm = functools.partial(
    pl.kernel,
    out_type=jax.ShapeDtypeStruct((NC, N_NODES, DH), jnp.float32),
    mesh=plsc.VectorSubcoreMesh(core_axis_name="c", subcore_axis_name="s",
                                num_cores=NC, num_subcores=NS),
    scratch_types=[
        pltpu.VMEM((NCHUNK, CHUNK), jnp.int32),
        pltpu.VMEM((NCHUNK, CHUNK), jnp.int32),
        pltpu.VMEM((EPT + 16,), jnp.float32),
        pltpu.VMEM((CHUNK, DH), jnp.float32),
        pltpu.VMEM_SHARED((N_NODES, DH), jnp.float32),
        pltpu.SemaphoreType.DMA,
    ],
    compiler_params=pltpu.CompilerParams(use_tc_tiling_on_sc=False),
)(_sc_body)


def kernel(adj_edge_index, adj_values, embeds):
    row = adj_edge_index[0].reshape(NS, NCHUNK, CHUNK)
    col = adj_edge_index[1].reshape(NS, NCHUNK, CHUNK)
    emb = embeds.reshape(N_NODES, NC, DH).transpose(1, 0, 2).reshape(NC * N_NODES, DH)
    out = _sc_spmm(row, col, adj_values, emb)
    return out.transpose(1, 0, 2).reshape(N_NODES, D_FEAT)


# speedup vs baseline: 9.2965x; 2.0265x over previous
import functools

import jax
import jax.numpy as jnp
from jax import lax
from jax.experimental import pallas as pl
from jax.experimental.pallas import tpu as pltpu
from jax.experimental.pallas import tpu_sc as plsc

N_NODES = 10000
N_EDGES = 320000
D_FEAT = 128

NC = 2
NS = 16
DH = D_FEAT // NC

CHUNK = 80
NB = 2
EPT = N_EDGES // NS
NCHUNK = EPT // CHUNK
NROUND = NCHUNK // NB
ROW_STEP = 624
ROW_SPAN = 640
NLANE = DH // 16


def _sc_body(row_hbm, col_hbm, val_hbm, emb_hbm, out_hbm,
             row_v, col_v, val_v, gbuf, sbuf, acc, sg, ss):
    c = lax.axis_index("c")
    s = lax.axis_index("s")

    pltpu.sync_copy(row_hbm.at[s], row_v)
    pltpu.sync_copy(col_hbm.at[s], col_v)
    pltpu.sync_copy(val_hbm.at[pl.ds(s * EPT, EPT)], val_v)

    cofs = jnp.full((16,), c * N_NODES, jnp.int32)

    def _bias(i, _):
        for t in range(CHUNK // 16):
            col_v[i, pl.ds(t * 16, 16)] = col_v[i, pl.ds(t * 16, 16)] + cofs
        return 0
    lax.fori_loop(0, NCHUNK, _bias, 0)

    def _zrow(i, _):
        for j in range(NLANE):
            gbuf[0, i, pl.ds(j * 16, 16)] = jnp.zeros((16,), jnp.float32)
        return 0
    lax.fori_loop(0, CHUNK, _zrow, 0)
    zbase = s * ROW_STEP
    for k in range(ROW_SPAN // CHUNK):
        pltpu.sync_copy(gbuf.at[0], acc.at[pl.ds(zbase + k * CHUNK, CHUNK)])
    plsc.subcore_barrier()

    def _gather_start(i, b):
        pltpu.async_copy(emb_hbm.at[col_v.at[i]], gbuf.at[b], sg.at[b])

    def _gather_wait(b):
        pltpu.make_async_copy(emb_hbm.at[col_v.at[0]], gbuf.at[b],
                              sg.at[b]).wait()

    def _scatter_start(i, b):
        pltpu.async_copy(sbuf.at[b], acc.at[row_v.at[i]], ss.at[b], add=True)

    def _scatter_wait(b):
        pltpu.make_async_copy(sbuf.at[b], acc.at[row_v.at[0]], ss.at[b]).wait()

    def _scale(i, b):
        def _group(g, _):
            e0 = g * 16
            vv = val_v[pl.ds(i * CHUNK + e0, 16)]
            for el in range(16):
                vs = jnp.full((16,), vv[el], jnp.float32)
                for j in range(NLANE):
                    sbuf[b, e0 + el, pl.ds(j * 16, 16)] = (
                        gbuf[b, e0 + el, pl.ds(j * 16, 16)] * vs)
            return 0
        lax.fori_loop(0, CHUNK // 16, _group, 0)

    for b in range(NB):
        _gather_start(b, b)

    for b in range(NB):
        _gather_wait(b)
        _scale(b, b)
        _scatter_start(b, b)
        _gather_start(NB + b, b)

    def _round(r, _):
        for b in range(NB):
            i = r * NB + b
            _gather_wait(b)
            _scatter_wait(b)
            _scale(i, b)
            _scatter_start(i, b)
            nxt = i + NB
            nxt = jnp.where(nxt < NCHUNK, nxt, NCHUNK - 1)
            _gather_start(nxt, b)
        return 0
    lax.fori_loop(1, NROUND, _round, 0)

    for b in range(NB):
        _gather_wait(b)
        _scatter_wait(b)
    plsc.subcore_barrier()

    pltpu.sync_copy(acc.at[pl.ds(s * ROW_STEP, ROW_SPAN)],
                    out_hbm.at[c, pl.ds(s * ROW_STEP, ROW_SPAN)])


_sc_spmm = functools.partial(
    pl.kernel,
    out_type=jax.ShapeDtypeStruct((NC, N_NODES, DH), jnp.float32),
    mesh=plsc.VectorSubcoreMesh(core_axis_name="c", subcore_axis_name="s",
                                num_cores=NC, num_subcores=NS),
    scratch_types=[
        pltpu.VMEM((NCHUNK, CHUNK), jnp.int32),
        pltpu.VMEM((NCHUNK, CHUNK), jnp.int32),
        pltpu.VMEM((EPT,), jnp.float32),
        pltpu.VMEM((NB, CHUNK, DH), jnp.float32),
        pltpu.VMEM((NB, CHUNK, DH), jnp.float32),
        pltpu.VMEM_SHARED((N_NODES, DH), jnp.float32),
        pltpu.SemaphoreType.DMA((NB,)),
        pltpu.SemaphoreType.DMA((NB,)),
    ],
    compiler_params=pltpu.CompilerParams(use_tc_tiling_on_sc=False),
)(_sc_body)


def kernel(adj_edge_index, adj_values, embeds):
    row = adj_edge_index[0].reshape(NS, NCHUNK, CHUNK)
    col = adj_edge_index[1].reshape(NS, NCHUNK, CHUNK)
    emb = embeds.reshape(N_NODES, NC, DH).transpose(1, 0, 2).reshape(NC * N_NODES, DH)
    out = _sc_spmm(row, col, adj_values, emb)
    return out.transpose(1, 0, 2).reshape(N_NODES, D_FEAT)
